# baseline (speedup 1.0000x reference)
import jax
import jax.numpy as jnp
from jax.experimental import pallas as pl

NUM_C = 7
ROW_BLK = 256


BATCH_BLK = 2


def _onehot_kernel(frame_ref, out_ref):
    rows = jax.lax.broadcasted_iota(jnp.int32, (ROW_BLK, 256), 0)
    loc_x = rows.astype(jnp.float32)
    loc_y = jax.lax.broadcasted_iota(jnp.int32, (ROW_BLK, 256), 1).astype(jnp.float32)
    for bb in range(BATCH_BLK):
        f = frame_ref[bb]
        for c in range(NUM_C):
            out_ref[bb, 3 * c] = loc_x
            out_ref[bb, 3 * c + 1] = loc_y
            out_ref[bb, 3 * c + 2] = (f == c).astype(jnp.float32)


def kernel(frame, embed_weights):
    del embed_weights
    B, H, W = frame.shape
    grid = (B // BATCH_BLK,)
    return pl.pallas_call(
        _onehot_kernel,
        grid=grid,
        in_specs=[pl.BlockSpec((BATCH_BLK, ROW_BLK, W), lambda b: (b, 0, 0))],
        out_specs=pl.BlockSpec((BATCH_BLK, 3 * NUM_C, ROW_BLK, W), lambda b: (b, 0, 0, 0)),
        out_shape=jax.ShapeDtypeStruct((B, 3 * NUM_C, H, W), jnp.float32),
    )(frame)
